# baseline (speedup 1.0000x reference)
import functools

import jax
import jax.numpy as jnp
from jax import lax
from jax.experimental import pallas as pl
from jax.experimental.pallas import tpu as pltpu
from jax.experimental.pallas import tpu_sc as plsc

H, W, P, D, EMB = 512, 512, 16, 128, 768
NR = H // P
NC = W // P
N_PATCH = NR * NC
LANES = 16
KSC = 8


def _sc_batch_add(inputs, row_emb, col_emb):
    mesh = plsc.VectorSubcoreMesh(core_axis_name="c", subcore_axis_name="s")

    @functools.partial(
        pl.kernel,
        mesh=mesh,
        out_type=jax.ShapeDtypeStruct((KSC, N_PATCH, EMB), jnp.float32),
        scratch_types=[
            pltpu.VMEM((NC,), jnp.int32),
            pltpu.VMEM((NC, EMB), jnp.float32),
            pltpu.VMEM((1, EMB), jnp.float32),
            pltpu.VMEM((2, NC, EMB), jnp.float32),
            pltpu.VMEM((2, NC, EMB), jnp.float32),
            pltpu.SemaphoreType.DMA,
            pltpu.SemaphoreType.DMA,
            pltpu.SemaphoreType.DMA,
            pltpu.SemaphoreType.DMA,
            pltpu.SemaphoreType.DMA,
        ],
    )
    def k(in_hbm, row_hbm, col_hbm, out_hbm, idx_v, pos_v, row_v, ibuf, obuf,
          sem_g, sem_i0, sem_i1, sem_o0, sem_o1):
        tid = lax.axis_index("s") * 2 + lax.axis_index("c")
        patch0 = tid * NC

        for half in range(NC // LANES):
            idx_v[pl.ds(half * LANES, LANES)] = (
                lax.iota(jnp.int32, LANES) + half * LANES
            ) * 4 + 2
        gather = pltpu.async_copy(col_hbm.at[idx_v], pos_v, sem_g)
        pltpu.sync_copy(row_hbm.at[pl.ds(tid * 4 + 2, 1)], row_v)
        gather.wait()

        def build(c, _):
            for j in range(EMB // LANES):
                sl = pl.ds(j * LANES, LANES)
                pos_v[c, sl] = pos_v[c, sl] + row_v[0, sl]
            return 0

        lax.fori_loop(0, NC, build, 0)

        sem_i = (sem_i0, sem_i1)
        sem_o = (sem_o0, sem_o1)
        in_dma = [None] * KSC
        out_dma = [None] * KSC
        in_dma[0] = pltpu.async_copy(
            in_hbm.at[0, pl.ds(patch0, NC)], ibuf.at[0], sem_i[0])
        for b in range(KSC):
            p = b & 1
            if b + 1 < KSC:
                in_dma[b + 1] = pltpu.async_copy(
                    in_hbm.at[b + 1, pl.ds(patch0, NC)], ibuf.at[1 - p],
                    sem_i[1 - p])
            in_dma[b].wait()
            if b >= 2:
                out_dma[b - 2].wait()

            def add(c, _):
                for j in range(EMB // LANES):
                    sl = pl.ds(j * LANES, LANES)
                    obuf[p, c, sl] = ibuf[p, c, sl] + pos_v[c, sl]
                return 0

            lax.fori_loop(0, NC, add, 0)
            out_dma[b] = pltpu.async_copy(
                obuf.at[p], out_hbm.at[b, pl.ds(patch0, NC)], sem_o[p])
        out_dma[KSC - 2].wait()
        out_dma[KSC - 1].wait()

    return k(inputs, row_emb, col_emb)


def _add_kernel(x_ref, row_ref, col_ref, out_ref, pos_ref):
    @pl.when(pl.program_id(0) == 0)
    def _build():
        r = row_ref[...].reshape(NR, D // NR, EMB)[:, 2, :]
        c = col_ref[...].reshape(NC, D // NC, EMB)[:, 2, :]
        pos = r[:, None, :] + c[None, :, :]
        pos_ref[...] = pos.reshape(N_PATCH, EMB)

    out_ref[...] = x_ref[...] + pos_ref[...][None, :, :]


@jax.jit
def kernel(inputs, row_embedding, col_embedding):
    B = inputs.shape[0]
    sc_out = _sc_batch_add(inputs, row_embedding, col_embedding)
    bb = 4
    ntc = B - KSC
    grid = (ntc // bb,)
    off = KSC // bb
    tc_out = pl.pallas_call(
        _add_kernel,
        grid=grid,
        in_specs=[
            pl.BlockSpec((bb, N_PATCH, EMB), lambda i: (i + off, 0, 0)),
            pl.BlockSpec((D, EMB), lambda i: (0, 0)),
            pl.BlockSpec((D, EMB), lambda i: (0, 0)),
        ],
        out_specs=pl.BlockSpec((bb, N_PATCH, EMB), lambda i: (i, 0, 0)),
        out_shape=jax.ShapeDtypeStruct((ntc, N_PATCH, EMB), inputs.dtype),
        scratch_shapes=[pltpu.VMEM((N_PATCH, EMB), jnp.float32)],
    )(inputs, row_embedding, col_embedding)
    return jnp.concatenate([sc_out, tc_out], axis=0)


# speedup vs baseline: 1.7916x; 1.7916x over previous
import functools

import jax
import jax.numpy as jnp
from jax import lax
from jax.experimental import pallas as pl
from jax.experimental.pallas import tpu as pltpu
from jax.experimental.pallas import tpu_sc as plsc

H, W, P, D, EMB = 512, 512, 16, 128, 768
NR = H // P
NC = W // P
N_PATCH = NR * NC
LANES = 16
BB = 4
K_HEAD = 8


def _pos_table_sc(row_emb, col_emb):
    mesh = plsc.VectorSubcoreMesh(core_axis_name="c", subcore_axis_name="s")

    @functools.partial(
        pl.kernel,
        mesh=mesh,
        out_type=jax.ShapeDtypeStruct((N_PATCH, EMB), jnp.float32),
        scratch_types=[
            pltpu.VMEM((NC,), jnp.int32),
            pltpu.VMEM((NC, EMB), jnp.float32),
            pltpu.VMEM((1, EMB), jnp.float32),
            pltpu.SemaphoreType.DMA,
        ],
    )
    def k(row_hbm, col_hbm, out_hbm, idx_v, cols_v, row_v, sem):
        tid = lax.axis_index("s") * 2 + lax.axis_index("c")
        for half in range(NC // LANES):
            idx_v[pl.ds(half * LANES, LANES)] = (
                lax.iota(jnp.int32, LANES) + half * LANES
            ) * 4 + 2
        gather = pltpu.async_copy(col_hbm.at[idx_v], cols_v, sem)
        pltpu.sync_copy(row_hbm.at[pl.ds(tid * 4 + 2, 1)], row_v)
        gather.wait()

        def build(c, _):
            for j in range(EMB // LANES):
                sl = pl.ds(j * LANES, LANES)
                cols_v[c, sl] = cols_v[c, sl] + row_v[0, sl]
            return 0

        lax.fori_loop(0, NC, build, 0)
        pltpu.sync_copy(cols_v, out_hbm.at[pl.ds(tid * NC, NC)])

    return k(row_emb, col_emb)


def _head_kernel(x_ref, row_ref, col_ref, out_ref, pos_ref):
    @pl.when(pl.program_id(0) == 0)
    def _build():
        r = row_ref[...].reshape(NR, D // NR, EMB)[:, 2, :]
        c = col_ref[...].reshape(NC, D // NC, EMB)[:, 2, :]
        pos_ref[...] = (r[:, None, :] + c[None, :, :]).reshape(N_PATCH, EMB)

    out_ref[...] = x_ref[...] + pos_ref[...][None, :, :]


def _tail_kernel(head_ref, x_ref, pos_ref, out_ref):
    del head_ref
    out_ref[...] = x_ref[...] + pos_ref[...][None, :, :]


@jax.jit
def kernel(inputs, row_embedding, col_embedding):
    B = inputs.shape[0]
    pos = _pos_table_sc(row_embedding, col_embedding)
    out_shape = jax.ShapeDtypeStruct(inputs.shape, inputs.dtype)
    head = pl.pallas_call(
        _head_kernel,
        grid=(K_HEAD // BB,),
        in_specs=[
            pl.BlockSpec((BB, N_PATCH, EMB), lambda i: (i, 0, 0)),
            pl.BlockSpec((D, EMB), lambda i: (0, 0)),
            pl.BlockSpec((D, EMB), lambda i: (0, 0)),
        ],
        out_specs=pl.BlockSpec((BB, N_PATCH, EMB), lambda i: (i, 0, 0)),
        out_shape=out_shape,
        scratch_shapes=[pltpu.VMEM((N_PATCH, EMB), jnp.float32)],
    )(inputs, row_embedding, col_embedding)
    off = K_HEAD // BB
    return pl.pallas_call(
        _tail_kernel,
        grid=((B - K_HEAD) // BB,),
        in_specs=[
            pl.BlockSpec(memory_space=pltpu.MemorySpace.HBM),
            pl.BlockSpec((BB, N_PATCH, EMB), lambda i: (i + off, 0, 0)),
            pl.BlockSpec((N_PATCH, EMB), lambda i: (0, 0)),
        ],
        out_specs=pl.BlockSpec((BB, N_PATCH, EMB), lambda i: (i + off, 0, 0)),
        out_shape=out_shape,
        input_output_aliases={0: 0},
    )(head, inputs, pos)


# speedup vs baseline: 1.8116x; 1.0112x over previous
import functools

import jax
import jax.numpy as jnp
from jax import lax
from jax.experimental import pallas as pl
from jax.experimental.pallas import tpu as pltpu
from jax.experimental.pallas import tpu_sc as plsc

H, W, P, D, EMB = 512, 512, 16, 128, 768
NR = H // P
NC = W // P
N_PATCH = NR * NC
LANES = 16
BB = 4
BB_HEAD = 2
K_HEAD = 8


def _pos_table_sc(row_emb, col_emb):
    mesh = plsc.VectorSubcoreMesh(core_axis_name="c", subcore_axis_name="s")

    @functools.partial(
        pl.kernel,
        mesh=mesh,
        out_type=jax.ShapeDtypeStruct((N_PATCH, EMB), jnp.float32),
        scratch_types=[
            pltpu.VMEM((NC,), jnp.int32),
            pltpu.VMEM((NC, EMB), jnp.float32),
            pltpu.VMEM((1, EMB), jnp.float32),
            pltpu.SemaphoreType.DMA,
        ],
    )
    def k(row_hbm, col_hbm, out_hbm, idx_v, cols_v, row_v, sem):
        tid = lax.axis_index("s") * 2 + lax.axis_index("c")
        for half in range(NC // LANES):
            idx_v[pl.ds(half * LANES, LANES)] = (
                lax.iota(jnp.int32, LANES) + half * LANES
            ) * 4 + 2
        gather = pltpu.async_copy(col_hbm.at[idx_v], cols_v, sem)
        pltpu.sync_copy(row_hbm.at[pl.ds(tid * 4 + 2, 1)], row_v)
        gather.wait()

        row_chunks = [
            row_v[0, pl.ds(j * LANES, LANES)] for j in range(EMB // LANES)
        ]

        def build(c, _):
            for j in range(EMB // LANES):
                sl = pl.ds(j * LANES, LANES)
                cols_v[c, sl] = cols_v[c, sl] + row_chunks[j]
            return 0

        lax.fori_loop(0, NC, build, 0)
        pltpu.sync_copy(cols_v, out_hbm.at[pl.ds(tid * NC, NC)])

    return k(row_emb, col_emb)


def _head_kernel(x_ref, row_ref, col_ref, out_ref, pos_ref):
    @pl.when(pl.program_id(0) == 0)
    def _build():
        r = row_ref[...].reshape(NR, D // NR, EMB)[:, 2, :]
        c = col_ref[...].reshape(NC, D // NC, EMB)[:, 2, :]
        pos_ref[...] = (r[:, None, :] + c[None, :, :]).reshape(N_PATCH, EMB)

    out_ref[...] = x_ref[...] + pos_ref[...][None, :, :]


def _tail_kernel(head_ref, x_ref, pos_ref, out_ref):
    del head_ref
    out_ref[...] = x_ref[...] + pos_ref[...][None, :, :]


@jax.jit
def kernel(inputs, row_embedding, col_embedding):
    B = inputs.shape[0]
    pos = _pos_table_sc(row_embedding, col_embedding)
    out_shape = jax.ShapeDtypeStruct(inputs.shape, inputs.dtype)
    head = pl.pallas_call(
        _head_kernel,
        grid=(K_HEAD // BB_HEAD,),
        in_specs=[
            pl.BlockSpec((BB_HEAD, N_PATCH, EMB), lambda i: (i, 0, 0)),
            pl.BlockSpec((D, EMB), lambda i: (0, 0)),
            pl.BlockSpec((D, EMB), lambda i: (0, 0)),
        ],
        out_specs=pl.BlockSpec((BB_HEAD, N_PATCH, EMB), lambda i: (i, 0, 0)),
        out_shape=out_shape,
        scratch_shapes=[pltpu.VMEM((N_PATCH, EMB), jnp.float32)],
    )(inputs, row_embedding, col_embedding)
    off = K_HEAD // BB
    return pl.pallas_call(
        _tail_kernel,
        grid=((B - K_HEAD) // BB,),
        in_specs=[
            pl.BlockSpec(memory_space=pltpu.MemorySpace.HBM),
            pl.BlockSpec((BB, N_PATCH, EMB), lambda i: (i + off, 0, 0)),
            pl.BlockSpec((N_PATCH, EMB), lambda i: (0, 0)),
        ],
        out_specs=pl.BlockSpec((BB, N_PATCH, EMB), lambda i: (i + off, 0, 0)),
        out_shape=out_shape,
        input_output_aliases={0: 0},
    )(head, inputs, pos)


# speedup vs baseline: 1.8382x; 1.0147x over previous
import functools

import jax
import jax.numpy as jnp
from jax import lax
from jax.experimental import pallas as pl
from jax.experimental.pallas import tpu as pltpu
from jax.experimental.pallas import tpu_sc as plsc

H, W, P, D, EMB = 512, 512, 16, 128, 768
NR = H // P
NC = W // P
N_PATCH = NR * NC
LANES = 16
BB = 4
BB_HEAD = 4
K_HEAD = 16


def _pos_table_sc(row_emb, col_emb):
    mesh = plsc.VectorSubcoreMesh(core_axis_name="c", subcore_axis_name="s")

    @functools.partial(
        pl.kernel,
        mesh=mesh,
        out_type=jax.ShapeDtypeStruct((N_PATCH, EMB), jnp.float32),
        scratch_types=[
            pltpu.VMEM((NC,), jnp.int32),
            pltpu.VMEM((NC, EMB), jnp.float32),
            pltpu.VMEM((1, EMB), jnp.float32),
            pltpu.SemaphoreType.DMA,
        ],
    )
    def k(row_hbm, col_hbm, out_hbm, idx_v, cols_v, row_v, sem):
        tid = lax.axis_index("s") * 2 + lax.axis_index("c")
        for half in range(NC // LANES):
            idx_v[pl.ds(half * LANES, LANES)] = (
                lax.iota(jnp.int32, LANES) + half * LANES
            ) * 4 + 2
        gather = pltpu.async_copy(col_hbm.at[idx_v], cols_v, sem)
        pltpu.sync_copy(row_hbm.at[pl.ds(tid * 4 + 2, 1)], row_v)
        gather.wait()

        row_chunks = [
            row_v[0, pl.ds(j * LANES, LANES)] for j in range(EMB // LANES)
        ]

        def build(c, _):
            for j in range(EMB // LANES):
                sl = pl.ds(j * LANES, LANES)
                cols_v[c, sl] = cols_v[c, sl] + row_chunks[j]
            return 0

        lax.fori_loop(0, NC, build, 0)
        pltpu.sync_copy(cols_v, out_hbm.at[pl.ds(tid * NC, NC)])

    return k(row_emb, col_emb)


def _head_kernel(x_ref, row_ref, col_ref, out_ref, pos_ref):
    @pl.when(pl.program_id(0) == 0)
    def _build():
        r = row_ref[...].reshape(NR, D // NR, EMB)[:, 2, :]
        c = col_ref[...].reshape(NC, D // NC, EMB)[:, 2, :]
        pos_ref[...] = (r[:, None, :] + c[None, :, :]).reshape(N_PATCH, EMB)

    out_ref[...] = x_ref[...] + pos_ref[...][None, :, :]


def _tail_kernel(head_ref, x_ref, pos_ref, out_ref):
    del head_ref
    out_ref[...] = x_ref[...] + pos_ref[...][None, :, :]


@jax.jit
def kernel(inputs, row_embedding, col_embedding):
    B = inputs.shape[0]
    pos = _pos_table_sc(row_embedding, col_embedding)
    out_shape = jax.ShapeDtypeStruct(inputs.shape, inputs.dtype)
    head = pl.pallas_call(
        _head_kernel,
        grid=(K_HEAD // BB_HEAD,),
        in_specs=[
            pl.BlockSpec((BB_HEAD, N_PATCH, EMB), lambda i: (i, 0, 0)),
            pl.BlockSpec((D, EMB), lambda i: (0, 0)),
            pl.BlockSpec((D, EMB), lambda i: (0, 0)),
        ],
        out_specs=pl.BlockSpec((BB_HEAD, N_PATCH, EMB), lambda i: (i, 0, 0)),
        out_shape=out_shape,
        scratch_shapes=[pltpu.VMEM((N_PATCH, EMB), jnp.float32)],
    )(inputs, row_embedding, col_embedding)
    off = K_HEAD // BB
    return pl.pallas_call(
        _tail_kernel,
        grid=((B - K_HEAD) // BB,),
        in_specs=[
            pl.BlockSpec(memory_space=pltpu.MemorySpace.HBM),
            pl.BlockSpec((BB, N_PATCH, EMB), lambda i: (i + off, 0, 0)),
            pl.BlockSpec((N_PATCH, EMB), lambda i: (0, 0)),
        ],
        out_specs=pl.BlockSpec((BB, N_PATCH, EMB), lambda i: (i + off, 0, 0)),
        out_shape=out_shape,
        input_output_aliases={0: 0},
    )(head, inputs, pos)


# speedup vs baseline: 1.8469x; 1.0047x over previous
import functools

import jax
import jax.numpy as jnp
from jax import lax
from jax.experimental import pallas as pl
from jax.experimental.pallas import tpu as pltpu
from jax.experimental.pallas import tpu_sc as plsc

H, W, P, D, EMB = 512, 512, 16, 128, 768
NR = H // P
NC = W // P
N_PATCH = NR * NC
LANES = 16
BB = 4
BB_HEAD = 4
K_HEAD = 32


def _pos_table_sc(row_emb, col_emb):
    mesh = plsc.VectorSubcoreMesh(core_axis_name="c", subcore_axis_name="s")

    @functools.partial(
        pl.kernel,
        mesh=mesh,
        out_type=jax.ShapeDtypeStruct((N_PATCH, EMB), jnp.float32),
        scratch_types=[
            pltpu.VMEM((NC,), jnp.int32),
            pltpu.VMEM((NC, EMB), jnp.float32),
            pltpu.VMEM((1, EMB), jnp.float32),
            pltpu.SemaphoreType.DMA,
        ],
    )
    def k(row_hbm, col_hbm, out_hbm, idx_v, cols_v, row_v, sem):
        tid = lax.axis_index("s") * 2 + lax.axis_index("c")
        for half in range(NC // LANES):
            idx_v[pl.ds(half * LANES, LANES)] = (
                lax.iota(jnp.int32, LANES) + half * LANES
            ) * 4 + 2
        gather = pltpu.async_copy(col_hbm.at[idx_v], cols_v, sem)
        pltpu.sync_copy(row_hbm.at[pl.ds(tid * 4 + 2, 1)], row_v)
        gather.wait()

        row_chunks = [
            row_v[0, pl.ds(j * LANES, LANES)] for j in range(EMB // LANES)
        ]

        def build(c, _):
            for j in range(EMB // LANES):
                sl = pl.ds(j * LANES, LANES)
                cols_v[c, sl] = cols_v[c, sl] + row_chunks[j]
            return 0

        lax.fori_loop(0, NC, build, 0)
        pltpu.sync_copy(cols_v, out_hbm.at[pl.ds(tid * NC, NC)])

    return k(row_emb, col_emb)


def _head_kernel(x_ref, row_ref, col_ref, out_ref, pos_ref):
    @pl.when(pl.program_id(0) == 0)
    def _build():
        r = row_ref[...].reshape(NR, D // NR, EMB)[:, 2, :]
        c = col_ref[...].reshape(NC, D // NC, EMB)[:, 2, :]
        pos_ref[...] = (r[:, None, :] + c[None, :, :]).reshape(N_PATCH, EMB)

    out_ref[...] = x_ref[...] + pos_ref[...][None, :, :]


def _tail_kernel(head_ref, x_ref, pos_ref, out_ref):
    del head_ref
    out_ref[...] = x_ref[...] + pos_ref[...][None, :, :]


@jax.jit
def kernel(inputs, row_embedding, col_embedding):
    B = inputs.shape[0]
    pos = _pos_table_sc(row_embedding, col_embedding)
    out_shape = jax.ShapeDtypeStruct(inputs.shape, inputs.dtype)
    head = pl.pallas_call(
        _head_kernel,
        grid=(K_HEAD // BB_HEAD,),
        in_specs=[
            pl.BlockSpec((BB_HEAD, N_PATCH, EMB), lambda i: (i, 0, 0)),
            pl.BlockSpec((D, EMB), lambda i: (0, 0)),
            pl.BlockSpec((D, EMB), lambda i: (0, 0)),
        ],
        out_specs=pl.BlockSpec((BB_HEAD, N_PATCH, EMB), lambda i: (i, 0, 0)),
        out_shape=out_shape,
        scratch_shapes=[pltpu.VMEM((N_PATCH, EMB), jnp.float32)],
    )(inputs, row_embedding, col_embedding)
    off = K_HEAD // BB
    return pl.pallas_call(
        _tail_kernel,
        grid=((B - K_HEAD) // BB,),
        in_specs=[
            pl.BlockSpec(memory_space=pltpu.MemorySpace.HBM),
            pl.BlockSpec((BB, N_PATCH, EMB), lambda i: (i + off, 0, 0)),
            pl.BlockSpec((N_PATCH, EMB), lambda i: (0, 0)),
        ],
        out_specs=pl.BlockSpec((BB, N_PATCH, EMB), lambda i: (i + off, 0, 0)),
        out_shape=out_shape,
        input_output_aliases={0: 0},
    )(head, inputs, pos)


# speedup vs baseline: 1.9350x; 1.0477x over previous
import functools

import jax
import jax.numpy as jnp
from jax import lax
from jax.experimental import pallas as pl
from jax.experimental.pallas import tpu as pltpu
from jax.experimental.pallas import tpu_sc as plsc

H, W, P, D, EMB = 512, 512, 16, 128, 768
NR = H // P
NC = W // P
N_PATCH = NR * NC
BB = 4
BB_HEAD = 4
K_HEAD = 8


def _lookup_rows_sc(row_emb, col_emb):
    mesh = plsc.VectorSubcoreMesh(core_axis_name="c", subcore_axis_name="s")

    @functools.partial(
        pl.kernel,
        mesh=mesh,
        out_type=(
            jax.ShapeDtypeStruct((NR, EMB), jnp.float32),
            jax.ShapeDtypeStruct((NC, EMB), jnp.float32),
        ),
        scratch_types=[
            pltpu.VMEM((1, EMB), jnp.float32),
            pltpu.VMEM((1, EMB), jnp.float32),
            pltpu.SemaphoreType.DMA,
            pltpu.SemaphoreType.DMA,
        ],
    )
    def k(row_hbm, col_hbm, rows_out, cols_out, r_v, c_v, sem_r, sem_c):
        tid = lax.axis_index("s") * 2 + lax.axis_index("c")
        src = pl.ds(tid * 4 + 2, 1)
        dst = pl.ds(tid, 1)
        ld_r = pltpu.async_copy(row_hbm.at[src], r_v, sem_r)
        ld_c = pltpu.async_copy(col_hbm.at[src], c_v, sem_c)
        ld_r.wait()
        ld_c.wait()
        st_r = pltpu.async_copy(r_v, rows_out.at[dst], sem_r)
        st_c = pltpu.async_copy(c_v, cols_out.at[dst], sem_c)
        st_r.wait()
        st_c.wait()

    return k(row_emb, col_emb)


def _head_kernel(x_ref, row_ref, col_ref, out_ref, pos_ref):
    @pl.when(pl.program_id(0) == 0)
    def _build():
        r = row_ref[...].reshape(NR, D // NR, EMB)[:, 2, :]
        c = col_ref[...].reshape(NC, D // NC, EMB)[:, 2, :]
        pos_ref[...] = (r[:, None, :] + c[None, :, :]).reshape(N_PATCH, EMB)

    out_ref[...] = x_ref[...] + pos_ref[...][None, :, :]


def _tail_kernel(head_ref, x_ref, r_ref, c_ref, out_ref, pos_ref):
    del head_ref

    @pl.when(pl.program_id(0) == 0)
    def _build():
        pos = r_ref[...][:, None, :] + c_ref[...][None, :, :]
        pos_ref[...] = pos.reshape(N_PATCH, EMB)

    out_ref[...] = x_ref[...] + pos_ref[...][None, :, :]


@jax.jit
def kernel(inputs, row_embedding, col_embedding):
    B = inputs.shape[0]
    rows, cols = _lookup_rows_sc(row_embedding, col_embedding)
    out_shape = jax.ShapeDtypeStruct(inputs.shape, inputs.dtype)
    head = pl.pallas_call(
        _head_kernel,
        grid=(K_HEAD // BB_HEAD,),
        in_specs=[
            pl.BlockSpec((BB_HEAD, N_PATCH, EMB), lambda i: (i, 0, 0)),
            pl.BlockSpec((D, EMB), lambda i: (0, 0)),
            pl.BlockSpec((D, EMB), lambda i: (0, 0)),
        ],
        out_specs=pl.BlockSpec((BB_HEAD, N_PATCH, EMB), lambda i: (i, 0, 0)),
        out_shape=out_shape,
        scratch_shapes=[pltpu.VMEM((N_PATCH, EMB), jnp.float32)],
    )(inputs, row_embedding, col_embedding)
    off = K_HEAD // BB
    return pl.pallas_call(
        _tail_kernel,
        grid=((B - K_HEAD) // BB,),
        in_specs=[
            pl.BlockSpec(memory_space=pltpu.MemorySpace.HBM),
            pl.BlockSpec((BB, N_PATCH, EMB), lambda i: (i + off, 0, 0)),
            pl.BlockSpec((NR, EMB), lambda i: (0, 0)),
            pl.BlockSpec((NC, EMB), lambda i: (0, 0)),
        ],
        out_specs=pl.BlockSpec((BB, N_PATCH, EMB), lambda i: (i + off, 0, 0)),
        out_shape=out_shape,
        input_output_aliases={0: 0},
        scratch_shapes=[pltpu.VMEM((N_PATCH, EMB), jnp.float32)],
    )(head, inputs, rows, cols)


# speedup vs baseline: 2.1859x; 1.1297x over previous
import jax
import jax.numpy as jnp
from jax.experimental import pallas as pl
from jax.experimental.pallas import tpu as pltpu

H, W, P, D, EMB = 512, 512, 16, 128, 768
NR = H // P
NC = W // P
N_PATCH = NR * NC
BB = 4


def _add_kernel(x_ref, row_ref, col_ref, out_ref, pos_ref):
    @pl.when(pl.program_id(0) == 0)
    def _build():
        r = row_ref[...].reshape(NR, D // NR, EMB)[:, 2, :]
        c = col_ref[...].reshape(NC, D // NC, EMB)[:, 2, :]
        pos = r[:, None, :] + c[None, :, :]
        pos_ref[...] = pos.reshape(N_PATCH, EMB)

    out_ref[...] = x_ref[...] + pos_ref[...][None, :, :]


@jax.jit
def kernel(inputs, row_embedding, col_embedding):
    B = inputs.shape[0]
    grid = (B // BB,)
    return pl.pallas_call(
        _add_kernel,
        grid=grid,
        in_specs=[
            pl.BlockSpec((BB, N_PATCH, EMB), lambda i: (i, 0, 0)),
            pl.BlockSpec((D, EMB), lambda i: (0, 0)),
            pl.BlockSpec((D, EMB), lambda i: (0, 0)),
        ],
        out_specs=pl.BlockSpec((BB, N_PATCH, EMB), lambda i: (i, 0, 0)),
        out_shape=jax.ShapeDtypeStruct(inputs.shape, inputs.dtype),
        scratch_shapes=[pltpu.VMEM((N_PATCH, EMB), jnp.float32)],
    )(inputs, row_embedding, col_embedding)
